# baseline (speedup 1.0000x reference)
import functools

import jax
import jax.numpy as jnp
from jax import lax
from jax.experimental import pallas as pl
from jax.experimental.pallas import tpu as pltpu
from jax.experimental.pallas import tpu_sc as plsc

N_E = 8192
E_DIM = 256
BETA = 0.25
B_TOK = 8192
BM = 256
N_BLOCKS = B_TOK // BM

N_WIN = 4
WIN = N_E // N_WIN

SC_CORES = 2
SC_SUBCORES = 16
SC_WORKERS = SC_CORES * SC_SUBCORES
ROWS_PER_WORKER = B_TOK // SC_WORKERS
GATHER_CHUNK = 128

GROUPS = B_TOK * N_E // 128
G_PER_WORKER = GROUPS // SC_WORKERS
FILL_ROWS = 512


def _round_bf16(v):
    u = lax.bitcast_convert_type(v, jnp.uint32)
    u = (u + jnp.uint32(0x7FFF) + ((u >> 16) & jnp.uint32(1))) \
        & jnp.uint32(0xFFFF0000)
    return lax.bitcast_convert_type(u, jnp.float32)


def _dist_body(zbf_ref, zsq_ref, ebf_ref, esq_ref,
               idx_ref, loss_ref, perp_ref, counts_ref, acc_ref):
    m = pl.program_id(0)
    zbf = zbf_ref[...]
    ebf = ebf_ref[...]
    mm = lax.dot_general(zbf, ebf, (((1,), (1,)), ((), ())),
                         preferred_element_type=jnp.float32)
    zsq = zsq_ref[...]
    esq = esq_ref[...]
    d = (zsq + esq) - 2.0 * mm
    big = jnp.int32(jnp.iinfo(jnp.int32).max)
    acc_v = jnp.full((BM, 1), jnp.inf, jnp.float32)
    sel_v = jnp.zeros((BM, 1), jnp.float32)
    acc_i = jnp.zeros((BM, 1), jnp.int32)
    for w in range(N_WIN):
        dw = d[:, w * WIN:(w + 1) * WIN]
        wmin = jnp.min(dw, axis=1, keepdims=True)
        jw = lax.broadcasted_iota(jnp.int32, dw.shape, 1) + jnp.int32(w * WIN)
        wi = jnp.min(jnp.where(dw == wmin, jw, big), axis=1, keepdims=True)
        lt = wmin < acc_v
        eq = wmin == acc_v
        acc_i = jnp.where(lt | (eq & (wi < acc_i)), wi, acc_i)
        sel_v = jnp.where(lt, wmin, sel_v)
        acc_v = _round_bf16(jnp.where(lt, wmin, acc_v))
    idx_ref[...] = acc_i

    part = jnp.sum(sel_v)
    bins = lax.broadcasted_iota(jnp.int32, (1, N_E), 1)
    cpart = jnp.sum((acc_i == bins).astype(jnp.float32), axis=0,
                    keepdims=True)

    @pl.when(m == 0)
    def _init():
        acc_ref[0, 0] = part
        counts_ref[...] = cpart

    @pl.when(m != 0)
    def _acc():
        acc_ref[0, 0] = acc_ref[0, 0] + part
        counts_ref[...] = counts_ref[...] + cpart

    @pl.when(m == jnp.int32(N_BLOCKS - 1))
    def _final():
        total = acc_ref[0, 0] + 0.0
        loss_ref[0, 0] = (1.0 + BETA) * total / jnp.float32(B_TOK * E_DIM)
        e_mean = counts_ref[...] * jnp.float32(1.0 / B_TOK)
        ent = jnp.sum(e_mean * jnp.log(e_mean + 1e-10))
        perp_ref[0, 0] = jnp.exp(-ent)


def _argmin_loss(zbf, zsq, ebf, esq):
    return pl.pallas_call(
        _dist_body,
        grid=(N_BLOCKS,),
        in_specs=[
            pl.BlockSpec((BM, E_DIM), lambda m: (m, 0)),
            pl.BlockSpec((BM, 1), lambda m: (m, 0)),
            pl.BlockSpec((N_E, E_DIM), lambda m: (0, 0)),
            pl.BlockSpec((1, N_E), lambda m: (0, 0)),
        ],
        out_specs=[
            pl.BlockSpec((BM, 1), lambda m: (m, 0)),
            pl.BlockSpec((1, 1), lambda m: (0, 0), memory_space=pltpu.SMEM),
            pl.BlockSpec((1, 1), lambda m: (0, 0), memory_space=pltpu.SMEM),
        ],
        out_shape=[
            jax.ShapeDtypeStruct((B_TOK, 1), jnp.int32),
            jax.ShapeDtypeStruct((1, 1), jnp.float32),
            jax.ShapeDtypeStruct((1, 1), jnp.float32),
        ],
        scratch_shapes=[
            pltpu.VMEM((1, N_E), jnp.float32),
            pltpu.SMEM((1, 1), jnp.float32),
        ],
    )(zbf, zsq, ebf, esq)


_SC_MESH = plsc.VectorSubcoreMesh(core_axis_name="c", subcore_axis_name="s")


def _fill_body(zeros_hbm, enc_hbm, zbuf, ):
    wid = lax.axis_index("s") * SC_CORES + lax.axis_index("c")
    base = wid * G_PER_WORKER
    pltpu.sync_copy(zeros_hbm, zbuf)

    def _chunk(c, carry):
        pltpu.sync_copy(zbuf,
                        enc_hbm.at[pl.ds(base + c * FILL_ROWS, FILL_ROWS), :])
        return carry

    lax.fori_loop(0, G_PER_WORKER // FILL_ROWS, _chunk, 0)


def _scatter_body(idx_hbm, ident_hbm, enc_hbm, idx_v, p_v, src_v, goff_v,
                  sem):
    wid = lax.axis_index("s") * SC_CORES + lax.axis_index("c")
    lanes = lax.iota(jnp.int32, 16)
    for c in range(ROWS_PER_WORKER // GATHER_CHUNK):
        base = wid * ROWS_PER_WORKER + c * GATHER_CHUNK
        pltpu.sync_copy(idx_hbm.at[pl.ds(base, GATHER_CHUNK)], idx_v)
        for t in range(GATHER_CHUNK // 16):
            kv = idx_v[pl.ds(t * 16, 16)]
            rows16 = lanes + jnp.int32(t * 16)
            p_v[pl.ds(t * 16, 16)] = kv & jnp.int32(127)
            goff_v[pl.ds(t * 16, 16)] = \
                (rows16 + jnp.int32(base)) * jnp.int32(N_E // 128) \
                + (kv >> jnp.int32(7))
        pltpu.async_copy(ident_hbm.at[p_v], src_v, sem).wait()
        pltpu.async_copy(src_v, enc_hbm.at[goff_v], sem).wait()


def _gather_kernel_body(idx_hbm, table_hbm, out_hbm, idx_v, rows_v, sem):
    wid = lax.axis_index("s") * SC_CORES + lax.axis_index("c")
    for c in range(ROWS_PER_WORKER // GATHER_CHUNK):
        base = wid * ROWS_PER_WORKER + c * GATHER_CHUNK
        pltpu.sync_copy(idx_hbm.at[pl.ds(base, GATHER_CHUNK)], idx_v)
        pltpu.async_copy(table_hbm.at[idx_v], rows_v, sem).wait()
        pltpu.sync_copy(rows_v, out_hbm.at[pl.ds(base, GATHER_CHUNK)])


def _sc_fill(zeros_small, enc_ref):
    call = functools.partial(
        pl.kernel,
        mesh=_SC_MESH,
        scratch_types=[pltpu.VMEM((FILL_ROWS, 128), jnp.float32)],
    )(_fill_body)
    call(zeros_small, enc_ref)


def _sc_scatter(idx_flat, ident, enc_ref):
    call = functools.partial(
        pl.kernel,
        mesh=_SC_MESH,
        scratch_types=[
            pltpu.VMEM((GATHER_CHUNK,), jnp.int32),
            pltpu.VMEM((GATHER_CHUNK,), jnp.int32),
            pltpu.VMEM((GATHER_CHUNK, 128), jnp.float32),
            pltpu.VMEM((GATHER_CHUNK,), jnp.int32),
            pltpu.SemaphoreType.DMA,
        ],
    )(_scatter_body)
    call(idx_flat, ident, enc_ref)


def _sc_gather(idx_flat, e):
    call = functools.partial(
        pl.kernel,
        out_type=jax.ShapeDtypeStruct((B_TOK, E_DIM), jnp.float32),
        mesh=_SC_MESH,
        scratch_types=[
            pltpu.VMEM((GATHER_CHUNK,), jnp.int32),
            pltpu.VMEM((GATHER_CHUNK, E_DIM), jnp.float32),
            pltpu.SemaphoreType.DMA,
        ],
    )(_gather_kernel_body)
    return call(idx_flat, e)


def _epi_body(z_ref, zq_ref, zqt_ref):
    zb = z_ref[0]
    zqb = zq_ref[...]
    zqt_ref[0] = (zb + (zqb - zb)).T


def _epilogue(z, z_q_rows):
    return pl.pallas_call(
        _epi_body,
        grid=(8,),
        in_specs=[
            pl.BlockSpec((1, 1024, E_DIM), lambda b: (b, 0, 0)),
            pl.BlockSpec((1024, E_DIM), lambda b: (b, 0)),
        ],
        out_specs=pl.BlockSpec((1, E_DIM, 1024), lambda b: (b, 0, 0)),
        out_shape=jax.ShapeDtypeStruct((8, E_DIM, 1024), jnp.float32),
    )(z, z_q_rows)


def kernel(z, embedding_weight):
    z_flat = z.reshape(-1, E_DIM)
    zsq = jnp.sum(z_flat ** 2, axis=1, keepdims=True)
    esq = jnp.sum(embedding_weight ** 2, axis=1)[None, :]
    zbf = z_flat.astype(jnp.bfloat16)
    ebf = embedding_weight.astype(jnp.bfloat16)

    enc_ref = jax.new_ref(lax.empty((GROUPS, 128), jnp.float32))
    zeros_small = jnp.zeros((FILL_ROWS, 128), jnp.float32)
    _sc_fill(zeros_small, enc_ref)

    idx2d, loss, perp = _argmin_loss(zbf, zsq, ebf, esq)
    idx_flat = idx2d.reshape(-1)
    ident = jnp.eye(128, dtype=jnp.float32)
    _sc_scatter(idx_flat, ident, enc_ref)
    min_encodings = jax.freeze(enc_ref).reshape(B_TOK, N_E)

    z_q_rows = _sc_gather(idx_flat, embedding_weight)
    z_q_t = _epilogue(z, z_q_rows)
    loss = loss.reshape(())
    perplexity = perp.reshape(())
    return (z_q_t, loss, (perplexity, min_encodings, idx2d))


# speedup vs baseline: 2.4076x; 2.4076x over previous
import functools

import jax
import jax.numpy as jnp
from jax import lax
from jax.experimental import pallas as pl
from jax.experimental.pallas import tpu as pltpu
from jax.experimental.pallas import tpu_sc as plsc

N_E = 8192
E_DIM = 256
BETA = 0.25
B_TOK = 8192
BM = 256
N_BLOCKS = B_TOK // BM

SC_CORES = 2
SC_SUBCORES = 16
SC_WORKERS = SC_CORES * SC_SUBCORES
ROWS_PER_WORKER = B_TOK // SC_WORKERS
GATHER_CHUNK = 128


N_WIN = 4
WIN = N_E // N_WIN


def _round_bf16(v):
    u = lax.bitcast_convert_type(v, jnp.uint32)
    u = (u + jnp.uint32(0x7FFF) + ((u >> 16) & jnp.uint32(1))) \
        & jnp.uint32(0xFFFF0000)
    return lax.bitcast_convert_type(u, jnp.float32)


def _dist_body(zbf_ref, zsq_ref, ebf_ref, esq_ref,
               idx_ref, enc_ref, loss_ref, perp_ref,
               counts_ref, acc_ref):
    m = pl.program_id(0)
    zbf = zbf_ref[...]
    ebf = ebf_ref[...]
    mm = lax.dot_general(zbf, ebf, (((1,), (1,)), ((), ())),
                         preferred_element_type=jnp.float32)
    zsq = zsq_ref[...]
    esq = esq_ref[...]
    d = (zsq + esq) - 2.0 * mm
    big = jnp.int32(jnp.iinfo(jnp.int32).max)
    jw0 = lax.broadcasted_iota(jnp.int32, (BM, WIN), 1)
    acc_v = jnp.full((BM, 1), jnp.inf, jnp.float32)
    sel_v = jnp.zeros((BM, 1), jnp.float32)
    acc_i = jnp.zeros((BM, 1), jnp.int32)
    for w in range(N_WIN):
        dw = d[:, w * WIN:(w + 1) * WIN]
        wmin = jnp.min(dw, axis=1, keepdims=True)
        wi = jnp.min(jnp.where(dw == wmin, jw0, big), axis=1, keepdims=True)
        wi = wi + jnp.int32(w * WIN)
        lt = wmin < acc_v
        eq = wmin == acc_v
        acc_i = jnp.where(lt | (eq & (wi < acc_i)), wi, acc_i)
        sel_v = jnp.where(lt, wmin, sel_v)
        acc_v = _round_bf16(jnp.where(lt, wmin, acc_v))
    idx_ref[...] = acc_i

    csums = []
    for w in range(N_WIN):
        enc_w = (jw0 == (acc_i - jnp.int32(w * WIN))).astype(jnp.float32)
        enc_ref[:, w * WIN:(w + 1) * WIN] = enc_w
        csums.append(jnp.sum(enc_w, axis=0, keepdims=True))
    cpart = jnp.concatenate(csums, axis=1)
    part = jnp.sum(sel_v)

    @pl.when(m == 0)
    def _init():
        acc_ref[0, 0] = part
        counts_ref[...] = cpart

    @pl.when(m != 0)
    def _acc():
        acc_ref[0, 0] = acc_ref[0, 0] + part
        counts_ref[...] = counts_ref[...] + cpart

    @pl.when(m == jnp.int32(N_BLOCKS - 1))
    def _final():
        total = acc_ref[0, 0] + 0.0
        loss_ref[0, 0] = (1.0 + BETA) * total / jnp.float32(B_TOK * E_DIM)
        e_mean = counts_ref[...] * jnp.float32(1.0 / B_TOK)
        ent = jnp.sum(e_mean * jnp.log(e_mean + 1e-10))
        perp_ref[0, 0] = jnp.exp(-ent)


def _argmin_onehot(zbf, zsq, ebf, esq):
    return pl.pallas_call(
        _dist_body,
        grid=(N_BLOCKS,),
        in_specs=[
            pl.BlockSpec((BM, E_DIM), lambda m: (m, 0)),
            pl.BlockSpec((BM, 1), lambda m: (m, 0)),
            pl.BlockSpec((N_E, E_DIM), lambda m: (0, 0)),
            pl.BlockSpec((1, N_E), lambda m: (0, 0)),
        ],
        out_specs=[
            pl.BlockSpec((BM, 1), lambda m: (m, 0)),
            pl.BlockSpec((BM, N_E), lambda m: (m, 0)),
            pl.BlockSpec((1, 1), lambda m: (0, 0), memory_space=pltpu.SMEM),
            pl.BlockSpec((1, 1), lambda m: (0, 0), memory_space=pltpu.SMEM),
        ],
        out_shape=[
            jax.ShapeDtypeStruct((B_TOK, 1), jnp.int32),
            jax.ShapeDtypeStruct((B_TOK, N_E), jnp.float32),
            jax.ShapeDtypeStruct((1, 1), jnp.float32),
            jax.ShapeDtypeStruct((1, 1), jnp.float32),
        ],
        scratch_shapes=[
            pltpu.VMEM((1, N_E), jnp.float32),
            pltpu.SMEM((1, 1), jnp.float32),
        ],
    )(zbf, zsq, ebf, esq)


def _gather_kernel_body(idx_hbm, table_hbm, out_hbm, idx_v, rows_v, sem):
    wid = lax.axis_index("s") * SC_CORES + lax.axis_index("c")
    for c in range(ROWS_PER_WORKER // GATHER_CHUNK):
        base = wid * ROWS_PER_WORKER + c * GATHER_CHUNK
        pltpu.sync_copy(idx_hbm.at[pl.ds(base, GATHER_CHUNK)], idx_v)
        pltpu.async_copy(table_hbm.at[idx_v], rows_v, sem).wait()
        pltpu.sync_copy(rows_v, out_hbm.at[pl.ds(base, GATHER_CHUNK)])


def _sc_gather(idx_flat, e):
    mesh = plsc.VectorSubcoreMesh(core_axis_name="c", subcore_axis_name="s")
    call = functools.partial(
        pl.kernel,
        out_type=jax.ShapeDtypeStruct((B_TOK, E_DIM), jnp.float32),
        mesh=mesh,
        scratch_types=[
            pltpu.VMEM((GATHER_CHUNK,), jnp.int32),
            pltpu.VMEM((GATHER_CHUNK, E_DIM), jnp.float32),
            pltpu.SemaphoreType.DMA,
        ],
    )(_gather_kernel_body)
    return call(idx_flat, e)


def _epi_body(z_ref, zq_ref, zqt_ref):
    zb = z_ref[0]
    zqb = zq_ref[...]
    zqt_ref[0] = (zb + (zqb - zb)).T


def _epilogue(z, z_q_rows):
    return pl.pallas_call(
        _epi_body,
        grid=(8,),
        in_specs=[
            pl.BlockSpec((1, 1024, E_DIM), lambda b: (b, 0, 0)),
            pl.BlockSpec((1024, E_DIM), lambda b: (b, 0)),
        ],
        out_specs=pl.BlockSpec((1, E_DIM, 1024), lambda b: (b, 0, 0)),
        out_shape=jax.ShapeDtypeStruct((8, E_DIM, 1024), jnp.float32),
    )(z, z_q_rows)


def kernel(z, embedding_weight):
    z_flat = z.reshape(-1, E_DIM)
    zsq = jnp.sum(z_flat ** 2, axis=1, keepdims=True)
    esq = jnp.sum(embedding_weight ** 2, axis=1)[None, :]
    zbf = z_flat.astype(jnp.bfloat16)
    ebf = embedding_weight.astype(jnp.bfloat16)
    idx2d, min_encodings, loss, perp = _argmin_onehot(zbf, zsq, ebf, esq)
    z_q_rows = _sc_gather(idx2d.reshape(-1), embedding_weight)
    z_q_t = _epilogue(z, z_q_rows)
    loss = loss.reshape(())
    perplexity = perp.reshape(())
    return (z_q_t, loss, (perplexity, min_encodings, idx2d))
